# baseline (speedup 1.0000x reference)
import functools

import jax
import jax.numpy as jnp
from jax import lax
from jax.experimental import pallas as pl
from jax.experimental.pallas import tpu as pltpu
from jax.experimental.pallas import tpu_sc as plsc

VOCAB = 1000
DPAD = 1024
N_ROWS = 1024 * 200
NC, NS = 2, 16
NW = NC * NS
ROWS_PER_W = N_ROWS // NW
CHUNK = 64
N_CHUNKS = ROWS_PER_W // CHUNK


def _gather_body(table_hbm, idx_hbm, out_hbm, idx_v, rows_v, sem):
    wid = lax.axis_index("s") * NC + lax.axis_index("c")
    base = wid * ROWS_PER_W

    def step(g, carry):
        off = base + g * CHUNK
        pltpu.sync_copy(idx_hbm.at[pl.ds(off, CHUNK)], idx_v)
        pltpu.async_copy(table_hbm.at[idx_v], rows_v, sem).wait()
        pltpu.sync_copy(rows_v, out_hbm.at[pl.ds(off, CHUNK), :])
        return carry

    lax.fori_loop(0, N_CHUNKS, step, 0)


@jax.jit
def kernel(idx, table):
    mesh = plsc.VectorSubcoreMesh(
        core_axis_name="c", subcore_axis_name="s", num_cores=NC, num_subcores=NS
    )
    k = functools.partial(
        pl.kernel,
        out_type=jax.ShapeDtypeStruct((N_ROWS, VOCAB), jnp.float32),
        mesh=mesh,
        scratch_types=[
            pltpu.VMEM((CHUNK,), jnp.int32),
            pltpu.VMEM((CHUNK, VOCAB), jnp.float32),
            pltpu.SemaphoreType.DMA,
        ],
        compiler_params=pltpu.CompilerParams(use_tc_tiling_on_sc=False),
    )(_gather_body)
    flat = k(table, idx.reshape(N_ROWS).astype(jnp.int32))
    return flat.reshape(idx.shape[0], idx.shape[1], VOCAB)


# speedup vs baseline: 1.5240x; 1.5240x over previous
import functools

import jax
import jax.numpy as jnp
from jax import lax
from jax.experimental import pallas as pl
from jax.experimental.pallas import tpu as pltpu
from jax.experimental.pallas import tpu_sc as plsc

VOCAB = 1000
DPAD = 1024
N_ROWS = 1024 * 200
NC, NS = 2, 16
NW = NC * NS
ROWS_PER_W = N_ROWS // NW
CHUNK = 64
N_CHUNKS = ROWS_PER_W // CHUNK
NT = DPAD // 128


TAIL = VOCAB - 896
TAIL_OFFS = (0, 16, 32, 48, 64, 80, TAIL - 16)


def _gather_body(table_hbm, idx_hbm, out_hbm, idx_v, rows_v, tail_v, sem):
    wid = lax.axis_index("s") * NC + lax.axis_index("c")
    base = wid * ROWS_PER_W

    def step(g, carry):
        off = base + g * CHUNK
        pltpu.sync_copy(idx_hbm.at[pl.ds(off, CHUNK)], idx_v)
        pltpu.async_copy(table_hbm.at[idx_v], rows_v, sem).wait()

        def repack(r, c2):
            for c in TAIL_OFFS:
                tail_v[r, pl.ds(c, 16)] = rows_v[r, pl.ds(896 + c, 16)]
            return c2

        lax.fori_loop(0, CHUNK, repack, 0)
        pltpu.sync_copy(
            rows_v.at[:, pl.ds(0, 896)],
            out_hbm.at[pl.ds(off, CHUNK), pl.ds(0, 896)],
        )
        pltpu.sync_copy(tail_v, out_hbm.at[pl.ds(off, CHUNK), pl.ds(896, TAIL)])
        return carry

    lax.fori_loop(0, N_CHUNKS, step, 0)


@jax.jit
def kernel(idx, table):
    mesh = plsc.VectorSubcoreMesh(
        core_axis_name="c", subcore_axis_name="s", num_cores=NC, num_subcores=NS
    )
    k = functools.partial(
        pl.kernel,
        out_type=jax.ShapeDtypeStruct((N_ROWS, VOCAB), jnp.float32),
        mesh=mesh,
        scratch_types=[
            pltpu.VMEM((CHUNK,), jnp.int32),
            pltpu.VMEM((CHUNK, DPAD), jnp.float32),
            pltpu.VMEM((CHUNK, TAIL), jnp.float32),
            pltpu.SemaphoreType.DMA,
        ],
    )(_gather_body)
    table_pad = jnp.pad(table, ((0, 0), (0, DPAD - VOCAB)))
    flat = k(table_pad, idx.reshape(N_ROWS).astype(jnp.int32))
    return flat.reshape(idx.shape[0], idx.shape[1], VOCAB)
